# baseline (speedup 1.0000x reference)
import jax
import jax.numpy as jnp
from jax.experimental import pallas as pl
from jax.experimental.pallas import tpu as pltpu

_N_EMB = 32
_B = 8
_C = 384
_H = 224
_W = 224
_CK = 8
_CSTEP = 128
_PER_STEP = _CSTEP // _CK
_NBUF = 8
_NTHREAD = 2
_CBLKS = _C // _CSTEP


def _body(x_hbm, tbl_ref, o_ref, ring, sems):
    bi = pl.program_id(0)
    cj = pl.program_id(1)
    e = jax.lax.broadcasted_iota(jnp.int32, (1, _N_EMB), 1)

    def issue(b, cstart, slot):
        pltpu.make_async_copy(
            x_hbm.at[b, pl.ds(cstart, _CK)], ring.at[slot], sems.at[slot]
        ).start(priority=slot % _NTHREAD)

    first = jnp.logical_and(bi == 0, cj == 0)
    last = jnp.logical_and(bi == _B - 1, cj == _CBLKS - 1)

    @pl.when(first)
    def _prime():
        for s in range(_NBUF):
            issue(jnp.int32(0), jnp.int32(s * _CK), s)

    for c in range(_PER_STEP):
        slot = c % _NBUF
        cstart = cj * _CSTEP + c * _CK
        pltpu.make_async_copy(
            x_hbm.at[bi, pl.ds(cstart, _CK)], ring.at[slot], sems.at[slot]
        ).wait()
        sums = jnp.sum(ring[slot], axis=(1, 2))
        mean = sums[:, None] / float(_H * _W)
        t = jnp.tanh(mean)
        idx = ((t + 1.0) / 2.0 * (_N_EMB - 1)).astype(jnp.int32)
        onehot = (idx == e).astype(jnp.float32)
        o_ref[c * _CK:(c + 1) * _CK, :] = jnp.sum(
            onehot * tbl_ref[...], axis=1, keepdims=True
        )
        nxt = cj * _CSTEP + (c + _NBUF) * _CK
        nb = jnp.where(nxt >= _C, bi + 1, bi)
        nc = jnp.where(nxt >= _C, nxt - _C, nxt)
        if c < _PER_STEP - _NBUF:
            issue(bi, cstart + _NBUF * _CK, slot)
        else:
            @pl.when(jnp.logical_not(last))
            def _refill():
                issue(nb, nc, slot)


def kernel(x, beta_table):
    b, c = x.shape[0], x.shape[1]
    tbl = beta_table.reshape(1, _N_EMB)
    out = pl.pallas_call(
        _body,
        grid=(_B, _CBLKS),
        in_specs=[
            pl.BlockSpec(memory_space=pltpu.MemorySpace.HBM),
            pl.BlockSpec((1, _N_EMB), lambda i, j: (0, 0)),
        ],
        out_specs=pl.BlockSpec((_CSTEP, 1), lambda i, j: (i * _CBLKS + j, 0)),
        out_shape=jax.ShapeDtypeStruct((_B * _C, 1), jnp.float32),
        scratch_shapes=[
            pltpu.VMEM((_NBUF, _CK, _H, _W), jnp.float32),
            pltpu.SemaphoreType.DMA((_NBUF,)),
        ],
        compiler_params=pltpu.CompilerParams(
            dimension_semantics=("arbitrary", "arbitrary"),
        ),
    )(x, tbl)
    return out.reshape(b, c, 1, 1)
